# Initial kernel scaffold; baseline (speedup 1.0000x reference)
import jax
import jax.numpy as jnp
from jax.experimental import pallas as pl


def kernel(hidden, y, emb_table, sampling_probs):
    raise NotImplementedError("write your pallas kernel here")


# speedup vs baseline: 1.2016x; 1.2016x over previous
import functools

import numpy as np
import jax
import jax.numpy as jnp
from jax.experimental import pallas as pl
from jax.experimental.pallas import tpu as pltpu

_VOCAB = 1000000
_D = 64
_NS = 2048
_BR = 1024

_GUMBEL = np.asarray(-jnp.log(-jnp.log(
    jax.random.uniform(jax.random.key(42), (_VOCAB,), minval=1e-10, maxval=1.0)
)))


def _loss_body(h_ref, epos_ref, yf_ref, tp_ref, eneg_ref, sid_ref, sp_ref,
               loss_ref, acc_ref, cnt_ref):
    step = pl.program_id(0)

    @pl.when(step == 0)
    def _():
        acc_ref[0, 0] = 0.0
        cnt_ref[0, 0] = 0.0

    h = h_ref[...]
    eneg = eneg_ref[...]
    neg = jax.lax.dot_general(
        h, eneg, (((1,), (1,)), ((), ())), preferred_element_type=jnp.float32
    )
    yf = yf_ref[...]
    sid = sid_ref[...]
    logsp = jnp.log(sp_ref[...] + 1e-10)
    negl = jnp.where(yf == sid, -1e9, neg) - logsp
    posl = (jnp.sum(h * epos_ref[...], axis=1, keepdims=True)
            - jnp.log(tp_ref[...] + 1e-10))
    m = jnp.maximum(jnp.max(negl, axis=1, keepdims=True), posl)
    s = jnp.sum(jnp.exp(negl - m), axis=1, keepdims=True) + jnp.exp(posl - m)
    per_row = m + jnp.log(s) - posl
    valid = yf != 0
    acc_ref[0, 0] += jnp.sum(jnp.where(valid, per_row, 0.0))
    cnt_ref[0, 0] += jnp.sum(valid.astype(jnp.float32))

    @pl.when(step == pl.num_programs(0) - 1)
    def _():
        loss_ref[...] = jnp.full((1, 1), acc_ref[0, 0] / cnt_ref[0, 0],
                                 dtype=jnp.float32)


def _fused_loss(h, epos, yf, tp, eneg, sid, sp):
    n = h.shape[0]
    grid = n // _BR
    return pl.pallas_call(
        _loss_body,
        grid=(grid,),
        in_specs=[
            pl.BlockSpec((_BR, _D), lambda i: (i, 0)),
            pl.BlockSpec((_BR, _D), lambda i: (i, 0)),
            pl.BlockSpec((_BR, 1), lambda i: (i, 0)),
            pl.BlockSpec((_BR, 1), lambda i: (i, 0)),
            pl.BlockSpec((_NS, _D), lambda i: (0, 0)),
            pl.BlockSpec((1, _NS), lambda i: (0, 0)),
            pl.BlockSpec((1, _NS), lambda i: (0, 0)),
        ],
        out_specs=pl.BlockSpec((1, 1), lambda i: (0, 0)),
        out_shape=jax.ShapeDtypeStruct((1, 1), jnp.float32),
        scratch_shapes=[
            pltpu.SMEM((1, 1), jnp.float32),
            pltpu.SMEM((1, 1), jnp.float32),
        ],
    )(h, epos, yf, tp, eneg, sid, sp)


def kernel(hidden, y, emb_table, sampling_probs):
    h = hidden.reshape(-1, _D)
    yf = y.reshape(-1).astype(jnp.int32)
    z = jnp.log(sampling_probs + 1e-10) + jnp.asarray(_GUMBEL)
    _, sid = jax.lax.top_k(z, _NS)
    sid = sid.astype(jnp.int32)
    epos = emb_table[yf]
    eneg = emb_table[sid]
    tp = sampling_probs[yf]
    sp = sampling_probs[sid]
    loss = _fused_loss(
        h, epos, yf.reshape(-1, 1), tp.reshape(-1, 1),
        eneg, sid.reshape(1, -1), sp.reshape(1, -1),
    )
    return loss[0, 0]


# speedup vs baseline: 2.7174x; 2.2615x over previous
import functools

import numpy as np
import jax
import jax.numpy as jnp
from jax.experimental import pallas as pl
from jax.experimental.pallas import tpu as pltpu

_VOCAB = 1000000
_D = 64
_NS = 2048
_BR = 1024

_GUMBEL = np.asarray(-jnp.log(-jnp.log(
    jax.random.uniform(jax.random.key(42), (_VOCAB,), minval=1e-10, maxval=1.0)
)))

_VPAD = 1024 * 1024
_GUMBEL_PAD = np.full((_VPAD,), -1e30, dtype=np.float32)
_GUMBEL_PAD[:_VOCAB] = _GUMBEL


_ROWS = 1024
_COLS = 1024
_NCHUNK = 32


def _select_body(probs_ref, gumbel_ref, keys_ref, offs_ref, tt_ref, ti_ref):
    z = jnp.log(probs_ref[...] + 1e-10) + gumbel_ref[...]
    b = jax.lax.bitcast_convert_type(z, jnp.uint32)
    key = jnp.where((b >> 31) == 1, ~b, b | jnp.uint32(0x80000000))
    keys_ref[...] = jax.lax.bitcast_convert_type(
        key ^ jnp.uint32(0x80000000), jnp.int32
    )

    row = jax.lax.broadcasted_iota(jnp.int32, (_ROWS, _COLS), 0)
    col = jax.lax.broadcasted_iota(jnp.int32, (_ROWS, _COLS), 1)
    idx = row * _COLS + col

    def bit_body(i, t):
        cand = t | jax.lax.shift_left(
            jnp.uint32(1), (31 - i).astype(jnp.uint32)
        )
        cnt = jnp.sum((key >= cand).astype(jnp.int32))
        return jnp.where(cnt >= _NS, cand, t)

    tval = jax.lax.fori_loop(0, 32, bit_body, jnp.uint32(0))

    need = _NS - jnp.sum((key > tval).astype(jnp.int32))

    def tie_body(j, iv):
        cand = iv | jax.lax.shift_left(jnp.int32(1), 19 - j)
        cnt = jnp.sum(((key == tval) & (idx <= cand)).astype(jnp.int32))
        return jnp.where(cnt <= need, cand, iv)

    ival = jax.lax.fori_loop(0, 20, tie_body, jnp.int32(0))

    sel = (key > tval) | ((key == tval) & (idx <= ival))
    rowsum = jnp.sum(sel.astype(jnp.float32), axis=1, keepdims=True)
    amat = (
        (jax.lax.broadcasted_iota(jnp.int32, (_NCHUNK, _ROWS), 1)
         // (_ROWS // _NCHUNK))
        == jax.lax.broadcasted_iota(jnp.int32, (_NCHUNK, _ROWS), 0)
    ).astype(jnp.float32)
    counts = jnp.dot(amat, rowsum, preferred_element_type=jnp.float32)
    ltri = (
        jax.lax.broadcasted_iota(jnp.int32, (_NCHUNK, _NCHUNK), 0)
        > jax.lax.broadcasted_iota(jnp.int32, (_NCHUNK, _NCHUNK), 1)
    ).astype(jnp.float32)
    offs = jnp.dot(ltri, counts, preferred_element_type=jnp.float32)
    offs_ref[...] = offs.astype(jnp.int32)
    t_signed = jax.lax.bitcast_convert_type(
        tval ^ jnp.uint32(0x80000000), jnp.int32
    )
    tt_ref[...] = jnp.full((1, 1), t_signed, dtype=jnp.int32)
    ti_ref[...] = jnp.full((1, 1), ival, dtype=jnp.int32)


def _select(probs2d, gumbel2d):
    return pl.pallas_call(
        _select_body,
        grid=(1,),
        in_specs=[
            pl.BlockSpec((_ROWS, _COLS), lambda i: (0, 0)),
            pl.BlockSpec((_ROWS, _COLS), lambda i: (0, 0)),
        ],
        out_specs=[
            pl.BlockSpec((_ROWS, _COLS), lambda i: (0, 0)),
            pl.BlockSpec((_NCHUNK, 1), lambda i: (0, 0)),
            pl.BlockSpec((1, 1), lambda i: (0, 0)),
            pl.BlockSpec((1, 1), lambda i: (0, 0)),
        ],
        out_shape=[
            jax.ShapeDtypeStruct((_ROWS, _COLS), jnp.int32),
            jax.ShapeDtypeStruct((_NCHUNK, 1), jnp.int32),
            jax.ShapeDtypeStruct((1, 1), jnp.int32),
            jax.ShapeDtypeStruct((1, 1), jnp.int32),
        ],
    )(probs2d, gumbel2d)


def _loss_body(h_ref, epos_ref, yf_ref, tp_ref, eneg_ref, sid_ref, sp_ref,
               loss_ref, acc_ref, cnt_ref):
    step = pl.program_id(0)

    @pl.when(step == 0)
    def _():
        acc_ref[0, 0] = 0.0
        cnt_ref[0, 0] = 0.0

    h = h_ref[...]
    eneg = eneg_ref[...]
    neg = jax.lax.dot_general(
        h, eneg, (((1,), (1,)), ((), ())), preferred_element_type=jnp.float32
    )
    yf = yf_ref[...]
    sid = sid_ref[...]
    logsp = jnp.log(sp_ref[...] + 1e-10)
    negl = jnp.where(yf == sid, -1e9, neg) - logsp
    posl = (jnp.sum(h * epos_ref[...], axis=1, keepdims=True)
            - jnp.log(tp_ref[...] + 1e-10))
    m = jnp.maximum(jnp.max(negl, axis=1, keepdims=True), posl)
    s = jnp.sum(jnp.exp(negl - m), axis=1, keepdims=True) + jnp.exp(posl - m)
    per_row = m + jnp.log(s) - posl
    valid = yf != 0
    acc_ref[0, 0] += jnp.sum(jnp.where(valid, per_row, 0.0))
    cnt_ref[0, 0] += jnp.sum(valid.astype(jnp.float32))

    @pl.when(step == pl.num_programs(0) - 1)
    def _():
        loss_ref[...] = jnp.full((1, 1), acc_ref[0, 0] / cnt_ref[0, 0],
                                 dtype=jnp.float32)


def _fused_loss(h, epos, yf, tp, eneg, sid, sp):
    n = h.shape[0]
    grid = n // _BR
    return pl.pallas_call(
        _loss_body,
        grid=(grid,),
        in_specs=[
            pl.BlockSpec((_BR, _D), lambda i: (i, 0)),
            pl.BlockSpec((_BR, _D), lambda i: (i, 0)),
            pl.BlockSpec((_BR, 1), lambda i: (i, 0)),
            pl.BlockSpec((_BR, 1), lambda i: (i, 0)),
            pl.BlockSpec((_NS, _D), lambda i: (0, 0)),
            pl.BlockSpec((1, _NS), lambda i: (0, 0)),
            pl.BlockSpec((1, _NS), lambda i: (0, 0)),
        ],
        out_specs=pl.BlockSpec((1, 1), lambda i: (0, 0)),
        out_shape=jax.ShapeDtypeStruct((1, 1), jnp.float32),
        scratch_shapes=[
            pltpu.SMEM((1, 1), jnp.float32),
            pltpu.SMEM((1, 1), jnp.float32),
        ],
    )(h, epos, yf, tp, eneg, sid, sp)


def kernel(hidden, y, emb_table, sampling_probs):
    h = hidden.reshape(-1, _D)
    yf = y.reshape(-1).astype(jnp.int32)
    probs_pad = jnp.zeros((_VPAD,), jnp.float32).at[:_VOCAB].set(sampling_probs)
    keys, offs, tt, ti = _select(
        probs_pad.reshape(_ROWS, _COLS),
        jnp.asarray(_GUMBEL_PAD).reshape(_ROWS, _COLS),
    )
    kflat = keys.reshape(-1)
    iall = jnp.arange(_VPAD, dtype=jnp.int32)
    selmask = (kflat > tt[0, 0]) | ((kflat == tt[0, 0]) & (iall <= ti[0, 0]))
    sid = jnp.nonzero(selmask, size=_NS, fill_value=0)[0].astype(jnp.int32)
    epos = emb_table[yf]
    eneg = emb_table[sid]
    tp = sampling_probs[yf]
    sp = sampling_probs[sid]
    loss = _fused_loss(
        h, epos, yf.reshape(-1, 1), tp.reshape(-1, 1),
        eneg, sid.reshape(1, -1), sp.reshape(1, -1),
    )
    return loss[0, 0]
